# baseline (speedup 1.0000x reference)
import functools

import jax
import jax.numpy as jnp
from jax import lax
from jax.experimental import pallas as pl
from jax.experimental.pallas import tpu as pltpu
from jax.experimental.pallas import tpu_sc as plsc

N_NODES = 10000
N_EDGES = 320000
NB = 160
UNROLL = 2
NSTAGE = 4
SB = NB // NSTAGE
EPT = NB * 128
N_PAD = 10112
RPT = N_PAD // 16
ROW_BLK = 1000


def _segsum_body(nt, srcp_hbm, dstp_hbm, zeros_hbm, *rest):
    tables = rest[:nt]
    outs = rest[nt:2 * nt]
    src_v, dst_v, rows_v, acc_sh = rest[2 * nt:2 * nt + 4]
    gsems = rest[2 * nt + 4:2 * nt + 4 + UNROLL]
    cid = lax.axis_index("c")
    sid = lax.axis_index("s")

    @pl.when(cid == 0)
    def _():
        for t in range(nt):
            pltpu.sync_copy(zeros_hbm.at[pl.ds(sid * RPT, RPT)],
                            acc_sh.at[pl.ds(sid * RPT, RPT)])
            plsc.subcore_barrier()

            def body(k, carry):
                gs = [pltpu.async_copy(
                          tables[t].at[src_v.at[k * UNROLL + i]],
                          rows_v.at[i], gsems[i])
                      for i in range(UNROLL)]
                for i in range(UNROLL):
                    gs[i].wait()
                    pltpu.sync_copy(rows_v.at[i],
                                    acc_sh.at[dst_v.at[k * UNROLL + i]],
                                    add=True)
                return carry

            for st in range(NSTAGE):
                pltpu.sync_copy(srcp_hbm.at[sid, pl.ds(st * SB, SB)], src_v)
                pltpu.sync_copy(dstp_hbm.at[sid, pl.ds(st * SB, SB)], dst_v)
                lax.fori_loop(0, SB // UNROLL, body, 0)
            plsc.subcore_barrier()
            pltpu.sync_copy(acc_sh.at[pl.ds(sid * RPT, RPT)],
                            outs[t].at[pl.ds(sid * RPT, RPT)])


def _segsum_sc(tables, srcp, dstp, zeros):
    nt = len(tables)
    mesh = plsc.VectorSubcoreMesh(core_axis_name="c", subcore_axis_name="s")
    out_type = tuple(jax.ShapeDtypeStruct((N_PAD, 128), jnp.float32)
                     for _ in range(nt))
    k = functools.partial(
        pl.kernel,
        out_type=out_type,
        mesh=mesh,
        scratch_types=[
            pltpu.VMEM((SB, 128), jnp.int32),
            pltpu.VMEM((SB, 128), jnp.int32),
            pltpu.VMEM((UNROLL, 128, 128), jnp.float32),
            pltpu.VMEM_SHARED((N_PAD, 128), jnp.float32),
        ] + [pltpu.SemaphoreType.DMA] * UNROLL,
    )(functools.partial(_segsum_body, nt))
    return k(srcp, dstp, zeros, *tables)


def _tc1_body(a_ref, W_ref, b_ref, out_ref):
    out_ref[0] = jnp.maximum(
        jnp.dot(a_ref[...], W_ref[...], preferred_element_type=jnp.float32)
        + b_ref[...], 0.0)


def _tc1(agg1, W1, b1):
    return pl.pallas_call(
        _tc1_body,
        grid=(N_NODES // ROW_BLK, 4),
        in_specs=[
            pl.BlockSpec((ROW_BLK, 128), lambda i, j: (i, 0)),
            pl.BlockSpec((128, 128), lambda i, j: (0, j)),
            pl.BlockSpec((1, 128), lambda i, j: (0, j)),
        ],
        out_specs=pl.BlockSpec((1, ROW_BLK, 128), lambda i, j: (j, i, 0)),
        out_shape=jax.ShapeDtypeStruct((4, N_NODES, 128), jnp.float32),
    )(agg1, W1, b1.reshape(1, 512))


def _tc2_body(p0_ref, p1_ref, p2_ref, p3_ref, W2_ref, b2_ref, Wf0_ref,
              bf0_ref, Wf1_ref, bf1_ref, Wf2_ref, bf2_ref, Wout_ref,
              bout_ref, out_ref):
    acc = jnp.broadcast_to(b2_ref[...], (p0_ref.shape[0], 512))
    for c, p_ref in enumerate((p0_ref, p1_ref, p2_ref, p3_ref)):
        acc = acc + jnp.dot(p_ref[...], W2_ref[pl.ds(c * 128, 128), :],
                            preferred_element_type=jnp.float32)
    h = jnp.maximum(acc, 0.0)
    h = jnp.maximum(jnp.dot(h, Wf0_ref[...], preferred_element_type=jnp.float32) + bf0_ref[...], 0.0)
    h = jnp.maximum(jnp.dot(h, Wf1_ref[...], preferred_element_type=jnp.float32) + bf1_ref[...], 0.0)
    h = jnp.maximum(jnp.dot(h, Wf2_ref[...], preferred_element_type=jnp.float32) + bf2_ref[...], 0.0)
    out_ref[...] = jnp.dot(h, Wout_ref[...], preferred_element_type=jnp.float32) + bout_ref[...]


def _tc2(parts, W2, b2, Wf0, bf0, Wf1, bf1, Wf2, bf2, Wout, bout):
    full = lambda shape: pl.BlockSpec(shape, lambda i: tuple(0 for _ in shape))
    part_spec = pl.BlockSpec((ROW_BLK, 128), lambda i: (i, 0))
    return pl.pallas_call(
        _tc2_body,
        grid=(N_NODES // ROW_BLK,),
        in_specs=[part_spec, part_spec, part_spec, part_spec,
                  full((512, 512)), full((1, 512)),
                  full((512, 512)), full((1, 512)),
                  full((512, 512)), full((1, 512)),
                  full((512, 512)), full((1, 512)),
                  full((512, 2)), full((1, 2))],
        out_specs=pl.BlockSpec((ROW_BLK, 2), lambda i: (i, 0)),
        out_shape=jax.ShapeDtypeStruct((N_NODES, 2), jnp.float32),
    )(*parts, W2, b2.reshape(1, 512), Wf0, bf0.reshape(1, 512),
      Wf1, bf1.reshape(1, 512), Wf2, bf2.reshape(1, 512),
      Wout, bout.reshape(1, 2))


def kernel(x, edge_index, W1, b1, W2, b2, Wf0, bf0, Wf1, bf1, Wf2, bf2, Wout, bout):
    ei = edge_index.astype(jnp.int32)
    ppt = EPT - N_EDGES // 16
    psrc = jnp.zeros((16, ppt), jnp.int32)
    pdst = jnp.broadcast_to(
        N_NODES + jnp.arange(ppt, dtype=jnp.int32) % (N_PAD - N_NODES),
        (16, ppt))
    srcp = jnp.concatenate([ei[0].reshape(16, -1), psrc], axis=1)
    srcp = srcp.reshape(16, NB, 128)
    dstp = jnp.concatenate([ei[1].reshape(16, -1), pdst], axis=1)
    dstp = dstp.reshape(16, NB, 128)
    zeros = jnp.zeros((N_PAD, 128), jnp.float32)

    (agg1,) = _segsum_sc([x], srcp, dstp, zeros)
    h1 = _tc1(agg1[:N_NODES], W1, b1)
    parts = _segsum_sc([h1[0], h1[1], h1[2], h1[3]], srcp, dstp, zeros)
    return _tc2([p[:N_NODES] for p in parts],
                W2, b2, Wf0, bf0, Wf1, bf1, Wf2, bf2, Wout, bout)


# speedup vs baseline: 1.6496x; 1.6496x over previous
import functools

import jax
import jax.numpy as jnp
from jax import lax
from jax.experimental import pallas as pl
from jax.experimental.pallas import tpu as pltpu
from jax.experimental.pallas import tpu_sc as plsc

N_NODES = 10000
N_EDGES = 320000
NB = 79
EPT = NB * 128
E_PAD = EPT * 32
N_PAD = 10112
RPT = N_PAD // 16
ROW_BLK = 1000


def _segsum_body(nt, srcp_hbm, dstp_hbm, zeros_hbm, *rest):
    tables = rest[:nt]
    outs = rest[nt:2 * nt]
    src_v, dst_v, rows_v, acc_sh, sem = rest[2 * nt:]
    cid = lax.axis_index("c")
    sid = lax.axis_index("s")
    wid = cid * 16 + sid
    pltpu.sync_copy(srcp_hbm.at[wid], src_v)
    pltpu.sync_copy(dstp_hbm.at[wid], dst_v)
    for t in range(nt):
        pltpu.sync_copy(zeros_hbm.at[pl.ds(sid * RPT, RPT)],
                        acc_sh.at[pl.ds(sid * RPT, RPT)])
        plsc.subcore_barrier()

        def body(j, carry):
            pltpu.async_copy(tables[t].at[src_v.at[j]], rows_v, sem).wait()
            pltpu.sync_copy(rows_v, acc_sh.at[dst_v.at[j]], add=True)
            return carry

        lax.fori_loop(0, NB, body, 0)
        plsc.subcore_barrier()
        pltpu.sync_copy(acc_sh.at[pl.ds(sid * RPT, RPT)],
                        outs[t].at[cid, pl.ds(sid * RPT, RPT)])


def _segsum_sc(tables, srcp, dstp, zeros):
    nt = len(tables)
    mesh = plsc.VectorSubcoreMesh(core_axis_name="c", subcore_axis_name="s")
    out_type = tuple(jax.ShapeDtypeStruct((2, N_PAD, 128), jnp.float32)
                     for _ in range(nt))
    k = functools.partial(
        pl.kernel,
        out_type=out_type,
        mesh=mesh,
        scratch_types=[
            pltpu.VMEM((NB, 128), jnp.int32),
            pltpu.VMEM((NB, 128), jnp.int32),
            pltpu.VMEM((128, 128), jnp.float32),
            pltpu.VMEM_SHARED((N_PAD, 128), jnp.float32),
            pltpu.SemaphoreType.DMA,
        ],
    )(functools.partial(_segsum_body, nt))
    return k(srcp, dstp, zeros, *tables)


def _tc1_body(a_ref, W_ref, b_ref, out_ref):
    s = a_ref[0] + a_ref[1]
    out_ref[0] = jnp.maximum(
        jnp.dot(s, W_ref[...], preferred_element_type=jnp.float32)
        + b_ref[...], 0.0)


def _tc1(agg1, W1, b1):
    return pl.pallas_call(
        _tc1_body,
        grid=(N_NODES // ROW_BLK, 4),
        in_specs=[
            pl.BlockSpec((2, ROW_BLK, 128), lambda i, j: (0, i, 0)),
            pl.BlockSpec((128, 128), lambda i, j: (0, j)),
            pl.BlockSpec((1, 128), lambda i, j: (0, j)),
        ],
        out_specs=pl.BlockSpec((1, ROW_BLK, 128), lambda i, j: (j, i, 0)),
        out_shape=jax.ShapeDtypeStruct((4, N_NODES, 128), jnp.float32),
    )(agg1, W1, b1.reshape(1, 512))


def _tc2_body(p0_ref, p1_ref, p2_ref, p3_ref, W2_ref, b2_ref, Wf0_ref,
              bf0_ref, Wf1_ref, bf1_ref, Wf2_ref, bf2_ref, Wout_ref,
              bout_ref, out_ref):
    acc = jnp.broadcast_to(b2_ref[...], (p0_ref.shape[1], 512))
    for c, p_ref in enumerate((p0_ref, p1_ref, p2_ref, p3_ref)):
        a = p_ref[0] + p_ref[1]
        acc = acc + jnp.dot(a, W2_ref[pl.ds(c * 128, 128), :],
                            preferred_element_type=jnp.float32)
    h = jnp.maximum(acc, 0.0)
    h = jnp.maximum(jnp.dot(h, Wf0_ref[...], preferred_element_type=jnp.float32) + bf0_ref[...], 0.0)
    h = jnp.maximum(jnp.dot(h, Wf1_ref[...], preferred_element_type=jnp.float32) + bf1_ref[...], 0.0)
    h = jnp.maximum(jnp.dot(h, Wf2_ref[...], preferred_element_type=jnp.float32) + bf2_ref[...], 0.0)
    out_ref[...] = jnp.dot(h, Wout_ref[...], preferred_element_type=jnp.float32) + bout_ref[...]


def _tc2(parts, W2, b2, Wf0, bf0, Wf1, bf1, Wf2, bf2, Wout, bout):
    full = lambda shape: pl.BlockSpec(shape, lambda i: tuple(0 for _ in shape))
    part_spec = pl.BlockSpec((2, ROW_BLK, 128), lambda i: (0, i, 0))
    return pl.pallas_call(
        _tc2_body,
        grid=(N_NODES // ROW_BLK,),
        in_specs=[part_spec, part_spec, part_spec, part_spec,
                  full((512, 512)), full((1, 512)),
                  full((512, 512)), full((1, 512)),
                  full((512, 512)), full((1, 512)),
                  full((512, 512)), full((1, 512)),
                  full((512, 2)), full((1, 2))],
        out_specs=pl.BlockSpec((ROW_BLK, 2), lambda i: (i, 0)),
        out_shape=jax.ShapeDtypeStruct((N_NODES, 2), jnp.float32),
    )(*parts, W2, b2.reshape(1, 512), Wf0, bf0.reshape(1, 512),
      Wf1, bf1.reshape(1, 512), Wf2, bf2.reshape(1, 512),
      Wout, bout.reshape(1, 2))


def kernel(x, edge_index, W1, b1, W2, b2, Wf0, bf0, Wf1, bf1, Wf2, bf2, Wout, bout):
    ei = edge_index.astype(jnp.int32)
    pad = E_PAD - N_EDGES
    srcp = jnp.concatenate([ei[0], jnp.zeros((pad,), jnp.int32)])
    srcp = srcp.reshape(32, NB, 128)
    dstp = jnp.concatenate([ei[1], jnp.full((pad,), N_NODES, jnp.int32)])
    dstp = dstp.reshape(32, NB, 128)
    zeros = jnp.zeros((N_PAD, 128), jnp.float32)

    (agg1,) = _segsum_sc([x], srcp, dstp, zeros)
    h1 = _tc1(agg1, W1, b1)
    parts = _segsum_sc([h1[0], h1[1], h1[2], h1[3]], srcp, dstp, zeros)
    return _tc2(parts, W2, b2, Wf0, bf0, Wf1, bf1, Wf2, bf2, Wout, bout)


# speedup vs baseline: 1.8279x; 1.1081x over previous
import functools

import jax
import jax.numpy as jnp
from jax import lax
from jax.experimental import pallas as pl
from jax.experimental.pallas import tpu as pltpu
from jax.experimental.pallas import tpu_sc as plsc

N_NODES = 10000
N_EDGES = 320000
NB = 79
EPT = NB * 128
E_PAD = EPT * 32
N_PAD = 10112
RPT = N_PAD // 16
ROW_BLK = 1000


def _segsum_body(nt, srcp_hbm, dstp_hbm, zeros_hbm, *rest):
    tables = rest[:nt]
    outs = rest[nt:2 * nt]
    src_v, dst_v, rows_v, acc_sh, sem = rest[2 * nt:]
    cid = lax.axis_index("c")
    sid = lax.axis_index("s")
    wid = cid * 16 + sid
    pltpu.sync_copy(srcp_hbm.at[wid], src_v)
    pltpu.sync_copy(dstp_hbm.at[wid], dst_v)
    for t in range(nt):
        pltpu.sync_copy(zeros_hbm.at[pl.ds(sid * RPT, RPT)],
                        acc_sh.at[pl.ds(sid * RPT, RPT)])
        plsc.subcore_barrier()

        def body(j, carry):
            pltpu.async_copy(tables[t].at[src_v.at[j]], rows_v, sem).wait()
            pltpu.sync_copy(rows_v, acc_sh.at[dst_v.at[j]], add=True)
            return carry

        lax.fori_loop(0, NB, body, 0)
        plsc.subcore_barrier()
        pltpu.sync_copy(acc_sh.at[pl.ds(sid * RPT, RPT)],
                        outs[t].at[cid, pl.ds(sid * RPT, RPT)])


def _segsum_sc(tables, srcp, dstp, zeros):
    nt = len(tables)
    mesh = plsc.VectorSubcoreMesh(core_axis_name="c", subcore_axis_name="s")
    out_type = tuple(jax.ShapeDtypeStruct((2, N_PAD, 128), jnp.float32)
                     for _ in range(nt))
    k = functools.partial(
        pl.kernel,
        out_type=out_type,
        mesh=mesh,
        scratch_types=[
            pltpu.VMEM((NB, 128), jnp.int32),
            pltpu.VMEM((NB, 128), jnp.int32),
            pltpu.VMEM((128, 128), jnp.float32),
            pltpu.VMEM_SHARED((N_PAD, 128), jnp.float32),
            pltpu.SemaphoreType.DMA,
        ],
    )(functools.partial(_segsum_body, nt))
    return k(srcp, dstp, zeros, *tables)


def _tc1_body(a_ref, W_ref, b_ref, out_ref):
    s = a_ref[0] + a_ref[1]
    out_ref[0] = jnp.maximum(
        jnp.dot(s, W_ref[...], preferred_element_type=jnp.float32)
        + b_ref[...], 0.0)


def _tc1(agg1, W1, b1):
    return pl.pallas_call(
        _tc1_body,
        grid=(N_NODES // ROW_BLK, 4),
        in_specs=[
            pl.BlockSpec((2, ROW_BLK, 128), lambda i, j: (0, i, 0)),
            pl.BlockSpec((128, 128), lambda i, j: (0, j)),
            pl.BlockSpec((1, 128), lambda i, j: (0, j)),
        ],
        out_specs=pl.BlockSpec((1, ROW_BLK, 128), lambda i, j: (j, i, 0)),
        out_shape=jax.ShapeDtypeStruct((4, N_NODES, 128), jnp.float32),
    )(agg1, W1, b1.reshape(1, 512))


def _tc2_body(p0_ref, p1_ref, p2_ref, p3_ref, W2_ref, b2_ref, Wf0_ref,
              bf0_ref, Wf1_ref, bf1_ref, Wf2_ref, bf2_ref, Wout_ref,
              bout_ref, out_ref):
    acc = jnp.broadcast_to(b2_ref[...], (p0_ref.shape[1], 512))
    for c, p_ref in enumerate((p0_ref, p1_ref, p2_ref, p3_ref)):
        a = p_ref[0] + p_ref[1]
        acc = acc + jnp.dot(a, W2_ref[pl.ds(c * 128, 128), :],
                            preferred_element_type=jnp.float32)
    h = jnp.maximum(acc, 0.0)
    h = jnp.maximum(jnp.dot(h, Wf0_ref[...], preferred_element_type=jnp.float32) + bf0_ref[...], 0.0)
    h = jnp.maximum(jnp.dot(h, Wf1_ref[...], preferred_element_type=jnp.float32) + bf1_ref[...], 0.0)
    h = jnp.maximum(jnp.dot(h, Wf2_ref[...], preferred_element_type=jnp.float32) + bf2_ref[...], 0.0)
    out_ref[...] = jnp.dot(h, Wout_ref[...], preferred_element_type=jnp.float32) + bout_ref[...]


def _tc2(parts, W2, b2, Wf0, bf0, Wf1, bf1, Wf2, bf2, Wout, bout):
    full = lambda shape: pl.BlockSpec(shape, lambda i: tuple(0 for _ in shape))
    part_spec = pl.BlockSpec((2, ROW_BLK, 128), lambda i: (0, i, 0))
    return pl.pallas_call(
        _tc2_body,
        grid=(N_NODES // ROW_BLK,),
        in_specs=[part_spec, part_spec, part_spec, part_spec,
                  full((512, 512)), full((1, 512)),
                  full((512, 512)), full((1, 512)),
                  full((512, 512)), full((1, 512)),
                  full((512, 512)), full((1, 512)),
                  full((512, 2)), full((1, 2))],
        out_specs=pl.BlockSpec((ROW_BLK, 2), lambda i: (i, 0)),
        out_shape=jax.ShapeDtypeStruct((N_NODES, 2), jnp.float32),
    )(*parts, W2, b2.reshape(1, 512), Wf0, bf0.reshape(1, 512),
      Wf1, bf1.reshape(1, 512), Wf2, bf2.reshape(1, 512),
      Wout, bout.reshape(1, 2))


def kernel(x, edge_index, W1, b1, W2, b2, Wf0, bf0, Wf1, bf1, Wf2, bf2, Wout, bout):
    ei = edge_index.astype(jnp.int32)
    ppt = EPT - N_EDGES // 32
    psrc = jnp.zeros((32, ppt), jnp.int32)
    pdst = jnp.broadcast_to(
        N_NODES + jnp.arange(ppt, dtype=jnp.int32) % (N_PAD - N_NODES),
        (32, ppt))
    srcp = jnp.concatenate([ei[0].reshape(32, -1), psrc], axis=1)
    srcp = srcp.reshape(32, NB, 128)
    dstp = jnp.concatenate([ei[1].reshape(32, -1), pdst], axis=1)
    dstp = dstp.reshape(32, NB, 128)
    zeros = jnp.zeros((N_PAD, 128), jnp.float32)

    (agg1,) = _segsum_sc([x], srcp, dstp, zeros)
    h1 = _tc1(agg1, W1, b1)
    parts = _segsum_sc([h1[0], h1[1], h1[2], h1[3]], srcp, dstp, zeros)
    return _tc2(parts, W2, b2, Wf0, bf0, Wf1, bf1, Wf2, bf2, Wout, bout)
